# baseline (speedup 1.0000x reference)
import jax
import jax.numpy as jnp
from jax.experimental import pallas as pl
from jax.experimental.pallas import tpu as pltpu

_BLOCK = 5000
_NSTEPS = 2
_SUBS = ((0, 1000), (1000, 1000), (2000, 1000), (3000, 1000), (4000, 1000))


def _linear_kernel(w_ref, b_ref, h_ref, out_hbm, out_vmem, sems):
    i = pl.program_id(0)
    for s, (off, size) in enumerate(_SUBS):
        rows = pl.ds(off, size)
        out_vmem[i, rows, :] = jax.lax.dot_general(
            h_ref[rows, :], w_ref[...],
            dimension_numbers=(((1,), (1,)), ((), ())),
            preferred_element_type=jnp.float32,
        ) + b_ref[...]
        pltpu.make_async_copy(
            out_vmem.at[i, rows, :],
            out_hbm.at[pl.ds(i * _BLOCK + off, size), :],
            sems.at[i, s],
        ).start()

    @pl.when(i == _NSTEPS - 1)
    def _drain():
        for ii in range(_NSTEPS):
            for s, (off, size) in enumerate(_SUBS):
                rows = pl.ds(off, size)
                pltpu.make_async_copy(
                    out_vmem.at[ii, rows, :],
                    out_hbm.at[pl.ds(ii * _BLOCK + off, size), :],
                    sems.at[ii, s],
                ).wait()


def kernel(h, edge_index, W, b):
    n, d_in = h.shape
    d_out = W.shape[0]
    return pl.pallas_call(
        _linear_kernel,
        grid=(_NSTEPS,),
        in_specs=[
            pl.BlockSpec(memory_space=pltpu.VMEM),
            pl.BlockSpec(memory_space=pltpu.VMEM),
            pl.BlockSpec((_BLOCK, d_in), lambda i: (i, 0)),
        ],
        out_specs=pl.BlockSpec(memory_space=pl.ANY),
        out_shape=jax.ShapeDtypeStruct((n, d_out), jnp.float32),
        scratch_shapes=[
            pltpu.VMEM((_NSTEPS, _BLOCK, d_out), jnp.float32),
            pltpu.SemaphoreType.DMA((_NSTEPS, len(_SUBS))),
        ],
        compiler_params=pltpu.CompilerParams(
            dimension_semantics=("arbitrary",),
        ),
    )(W, b.reshape(1, d_out), h)


# speedup vs baseline: 1.0746x; 1.0746x over previous
import jax
import jax.numpy as jnp
from jax.experimental import pallas as pl
from jax.experimental.pallas import tpu as pltpu

_BLOCK = 5000
_NSTEPS = 2
_SUBS = ((0, 1504), (1504, 1504), (3008, 1496), (4504, 496))


def _linear_kernel(w_ref, b_ref, h_ref, out_hbm, out_vmem, sems):
    i = pl.program_id(0)
    for s, (off, size) in enumerate(_SUBS):
        rows = pl.ds(off, size)
        out_vmem[i, rows, :] = jax.lax.dot_general(
            h_ref[rows, :], w_ref[...],
            dimension_numbers=(((1,), (1,)), ((), ())),
            preferred_element_type=jnp.float32,
        ) + b_ref[...]
        pltpu.make_async_copy(
            out_vmem.at[i, rows, :],
            out_hbm.at[pl.ds(i * _BLOCK + off, size), :],
            sems.at[i, s],
        ).start()

    @pl.when(i == _NSTEPS - 1)
    def _drain():
        for ii in range(_NSTEPS):
            for s, (off, size) in enumerate(_SUBS):
                rows = pl.ds(off, size)
                pltpu.make_async_copy(
                    out_vmem.at[ii, rows, :],
                    out_hbm.at[pl.ds(ii * _BLOCK + off, size), :],
                    sems.at[ii, s],
                ).wait()


def kernel(h, edge_index, W, b):
    n, d_in = h.shape
    d_out = W.shape[0]
    return pl.pallas_call(
        _linear_kernel,
        grid=(_NSTEPS,),
        in_specs=[
            pl.BlockSpec(memory_space=pltpu.VMEM),
            pl.BlockSpec(memory_space=pltpu.VMEM),
            pl.BlockSpec((_BLOCK, d_in), lambda i: (i, 0)),
        ],
        out_specs=pl.BlockSpec(memory_space=pl.ANY),
        out_shape=jax.ShapeDtypeStruct((n, d_out), jnp.float32),
        scratch_shapes=[
            pltpu.VMEM((_NSTEPS, _BLOCK, d_out), jnp.float32),
            pltpu.SemaphoreType.DMA((_NSTEPS, len(_SUBS))),
        ],
        compiler_params=pltpu.CompilerParams(
            dimension_semantics=("arbitrary",),
        ),
    )(W, b.reshape(1, d_out), h)
